# baseline (speedup 1.0000x reference)
import dataclasses
import functools

import jax
import jax.numpy as jnp
from jax import lax
from jax.experimental import pallas as pl
from jax.experimental.pallas import tpu as pltpu
from jax.experimental.pallas import tpu_sc as plsc

N = 10000
NP = 10240
E = 160000
NC, NS, LANES = 2, 16, 16
NW = NC * NS
NCH = 64
C = NP // NCH
CP = C + 8
CAP = 3072
EB = 4000
KA = 512
KB = 128
SENT_DST = 2 * N

_mesh = plsc.VectorSubcoreMesh(core_axis_name="c", subcore_axis_name="s")
_cp = pltpu.CompilerParams()
if "needs_layout_passes" in pltpu.CompilerParams.__dataclass_fields__:
    _cp = dataclasses.replace(_cp, needs_layout_passes=False)
if "use_tc_tiling_on_sc" in pltpu.CompilerParams.__dataclass_fields__:
    _cp = dataclasses.replace(_cp, use_tc_tiling_on_sc=False)

_IOTA = lambda: lax.iota(jnp.int32, LANES)


def _worker_id():
    return lax.axis_index("s") * NC + lax.axis_index("c")


def _sc_part(srcp, dstp):
    @functools.partial(
        pl.kernel,
        out_type=(
            jax.ShapeDtypeStruct((NCH, CAP), jnp.int32),
            jax.ShapeDtypeStruct((NCH, CAP), jnp.int32),
            jax.ShapeDtypeStruct((NCH * 16,), jnp.int32),
        ),
        mesh=_mesh,
        compiler_params=_cp,
        scratch_types=[
            pltpu.VMEM((EB,), jnp.int32),
            pltpu.VMEM((EB,), jnp.int32),
            pltpu.VMEM((CAP,), jnp.int32),
            pltpu.VMEM((CAP,), jnp.int32),
            pltpu.VMEM((CAP,), jnp.int32),
            pltpu.VMEM((CAP,), jnp.int32),
            pltpu.VMEM((16,), jnp.int32),
            pltpu.SemaphoreType.DMA,
            pltpu.SemaphoreType.DMA,
        ],
    )
    def k(srcp_hbm, dstp_hbm, src_c_hbm, dstloc_c_hbm, counts_hbm,
          dbuf0, dbuf1, eid0, eid1, srcv, dstv, cbuf, sem0, sem1):
        w = _worker_id()
        lo = w * 2 * C
        mid = lo + C
        hi = lo + 2 * C
        iota = _IOTA()
        nbat = E // EB

        def scan_batch(b, dbuf, carry):
            def group_body(g, ps):
                p0, p1 = ps
                d16 = dbuf[pl.ds(g * LANES, LANES)]
                eid16 = b * EB + g * LANES + iota
                mask0 = (d16 >= lo) & (d16 < mid)
                cum0 = plsc.cumsum(jnp.where(mask0, 1, 0))
                offs0 = jnp.minimum(p0 + cum0 - 1, CAP - 1)
                plsc.store_scatter(eid0, [offs0], eid16, mask=mask0)
                mask1 = (d16 >= mid) & (d16 < hi)
                cum1 = plsc.cumsum(jnp.where(mask1, 1, 0))
                offs1 = jnp.minimum(p1 + cum1 - 1, CAP - 1)
                plsc.store_scatter(eid1, [offs1], eid16, mask=mask1)
                return (jnp.minimum(p0 + cum0[15], CAP - 1),
                        jnp.minimum(p1 + cum1[15], CAP - 1))

            return lax.fori_loop(0, EB // LANES, group_body, carry)

        pltpu.async_copy(dstp_hbm.at[pl.ds(0, EB)], dbuf0, sem0)

        def batch_pair(q, carry):
            b0 = 2 * q
            b1 = b0 + 1

            pltpu.async_copy(dstp_hbm.at[pl.ds(b1 * EB, EB)], dbuf1, sem1)
            pltpu.make_async_copy(dstp_hbm.at[pl.ds(0, EB)], dbuf0, sem0).wait()
            carry = scan_batch(b0, dbuf0, carry)

            @pl.when(b1 + 1 < nbat)
            def _():
                pltpu.async_copy(dstp_hbm.at[pl.ds((b1 + 1) * EB, EB)],
                                 dbuf0, sem0)

            pltpu.make_async_copy(dstp_hbm.at[pl.ds(0, EB)], dbuf1, sem1).wait()
            carry = scan_batch(b1, dbuf1, carry)
            return carry

        cnt0, cnt1 = lax.fori_loop(0, nbat // 2, batch_pair, (0, 0))

        for cc, eidbuf, cnt, clo in ((0, eid0, cnt0, lo), (1, eid1, cnt1, mid)):
            chunk = w * 2 + cc

            @pl.loop(0, CAP // LANES)
            def _(g):
                pos = g * LANES + iota
                v = eidbuf[pl.ds(g * LANES, LANES)]
                eidbuf[pl.ds(g * LANES, LANES)] = jnp.where(pos >= cnt, E, v)

            pltpu.async_copy(srcp_hbm.at[eidbuf], srcv, sem0).wait()
            pltpu.async_copy(dstp_hbm.at[eidbuf], dstv, sem0).wait()

            @pl.loop(0, CAP // LANES)
            def _(g):
                v = dstv[pl.ds(g * LANES, LANES)]
                v = jnp.clip(jnp.minimum(v - clo, C), 0, C)
                dstv[pl.ds(g * LANES, LANES)] = v

            pltpu.sync_copy(srcv, src_c_hbm.at[chunk])
            pltpu.sync_copy(dstv, dstloc_c_hbm.at[chunk])
            cbuf[...] = jnp.where(iota == 0, cnt, 0)
            pltpu.sync_copy(cbuf, counts_hbm.at[pl.ds(chunk * 16, 16)])

    return k(srcp, dstp)


def _sc_attn(src_c, dstloc_c, counts, asrcT, adstT):
    @functools.partial(
        pl.kernel,
        out_type=(
            jax.ShapeDtypeStruct((NCH, 16, CAP), jnp.float32),
            jax.ShapeDtypeStruct((NP, 16), jnp.float32),
        ),
        mesh=_mesh,
        compiler_params=_cp,
        scratch_types=[
            pltpu.VMEM((16, CAP), jnp.float32),
            pltpu.VMEM((CP, 16), jnp.float32),
            pltpu.VMEM((CP, 16), jnp.float32),
            pltpu.VMEM((CP, 16), jnp.float32),
            pltpu.VMEM((KA, 16), jnp.float32),
            pltpu.VMEM((KA, 16), jnp.float32),
            pltpu.VMEM((CAP,), jnp.int32),
            pltpu.VMEM((CAP,), jnp.int32),
            pltpu.VMEM((16,), jnp.int32),
            pltpu.SemaphoreType.DMA,
            pltpu.SemaphoreType.DMA,
        ],
    )
    def k(src_c_hbm, dstloc_c_hbm, counts_hbm, asrcT_hbm, adstT_hbm,
          exT_hbm, den_hbm, eT, m, s, adst_loc, asg0, asg1, srcl, dstl,
          cbuf, sem0, sem1):
        w = _worker_id()
        iota = _IOTA()
        fz = jnp.zeros((LANES,), jnp.float32)

        for cc in range(2):
            chunk = w * 2 + cc
            pltpu.sync_copy(counts_hbm.at[pl.ds(chunk * 16, 16)], cbuf)
            cnt = cbuf[...][0]
            nb = (cnt + KA - 1) // KA

            pltpu.sync_copy(src_c_hbm.at[chunk], srcl)
            pltpu.sync_copy(dstloc_c_hbm.at[chunk], dstl)
            pltpu.sync_copy(adstT_hbm.at[pl.ds(chunk * C, C)],
                            adst_loc.at[pl.ds(0, C)])

            @pl.loop(0, CP)
            def _(r):
                m.at[r][...] = jnp.full((LANES,), -1e30, jnp.float32)
                s.at[r][...] = fz

            @pl.loop(C, CP)
            def _(r):
                adst_loc.at[r][...] = fz

            def p1_compute(b, asg):
                @pl.loop(0, KA // LANES)
                def _(g):
                    dl16 = dstl[pl.ds(b * KA + g * LANES, LANES)]
                    erows = []
                    for j in range(LANES):
                        i = g * LANES + j
                        e = asg.at[i][...] + adst_loc.at[dl16[j]][...]
                        erows.append(jnp.where(e > 0, e, 0.2 * e))
                    for j in range(LANES):
                        d = dl16[j]
                        m.at[d][...] = jnp.maximum(m.at[d][...], erows[j])
                    for j in range(LANES):
                        i = g * LANES + j
                        plsc.store_scatter(
                            eT, [iota, jnp.full((LANES,), b * KA + i,
                                                jnp.int32)], erows[j])

            def start(b, asg, sem):
                pltpu.async_copy(
                    asrcT_hbm.at[srcl.at[pl.ds(b * KA, KA)]], asg, sem)

            def wait(asg, sem):
                pltpu.make_async_copy(asrcT_hbm.at[pl.ds(0, KA)], asg,
                                      sem).wait()

            @pl.when(nb > 0)
            def _():
                start(0, asg0, sem0)

            def p1_pair(q, _):
                b0 = 2 * q
                b1 = b0 + 1

                @pl.when(b1 < nb)
                def _():
                    start(b1, asg1, sem1)

                wait(asg0, sem0)
                p1_compute(b0, asg0)

                @pl.when(b1 + 1 < nb)
                def _():
                    start(b1 + 1, asg0, sem0)

                @pl.when(b1 < nb)
                def _():
                    wait(asg1, sem1)
                    p1_compute(b1, asg1)
                return 0

            lax.fori_loop(0, (nb + 1) // 2, p1_pair, 0)

            def p2_batch(b, _):
                @pl.loop(0, KA // LANES)
                def _(g):
                    dl16 = dstl[pl.ds(b * KA + g * LANES, LANES)]
                    exs = []
                    for j in range(LANES):
                        i = g * LANES + j
                        colidx = jnp.full((LANES,), b * KA + i, jnp.int32)
                        erow = plsc.load_gather(eT, [iota, colidx])
                        exs.append(jnp.exp(erow - m.at[dl16[j]][...]))
                    for j in range(LANES):
                        plsc.addupdate(s.at[dl16[j]], exs[j])
                    for j in range(LANES):
                        i = g * LANES + j
                        plsc.store_scatter(
                            eT, [iota, jnp.full((LANES,), b * KA + i,
                                                jnp.int32)], exs[j])
                return 0

            lax.fori_loop(0, nb, p2_batch, 0)

            pltpu.sync_copy(s.at[pl.ds(0, C)],
                            den_hbm.at[pl.ds(chunk * C, C)])
            pltpu.sync_copy(eT, exT_hbm.at[chunk])

    return k(src_c, dstloc_c, counts, asrcT, adstT)


def _sc_aggr(H, src_c, dstloc_c, counts, exT, nheads, o):
    @functools.partial(
        pl.kernel,
        out_type=jax.ShapeDtypeStruct((nheads, NP, o), jnp.float32),
        mesh=_mesh,
        compiler_params=_cp,
        scratch_types=[
            pltpu.VMEM((CP, o), jnp.float32),
            pltpu.VMEM((KB, o), jnp.bfloat16),
            pltpu.VMEM((KB, o), jnp.bfloat16),
            pltpu.VMEM((CAP,), jnp.int32),
            pltpu.VMEM((CAP,), jnp.int32),
            pltpu.VMEM((CAP,), jnp.float32),
            pltpu.VMEM((16,), jnp.int32),
            pltpu.SemaphoreType.DMA,
            pltpu.SemaphoreType.DMA,
        ],
    )
    def k(H_hbm, src_c_hbm, dstloc_c_hbm, counts_hbm, exT_hbm, acc_hbm,
          acc, G0, G1, srcl, dstl, exl, cbuf, sem0, sem1):
        w = _worker_id()
        zseg = jnp.zeros((LANES,), jnp.float32)
        nseg = o // LANES
        nsg2 = o // 32

        @pl.loop(0, 2)
        def _(cc):
            chunk = w * 2 + cc
            pltpu.sync_copy(counts_hbm.at[pl.ds(chunk * 16, 16)], cbuf)
            cnt = cbuf[...][0]
            nb = (cnt + KB - 1) // KB
            pltpu.sync_copy(src_c_hbm.at[chunk], srcl)
            pltpu.sync_copy(dstloc_c_hbm.at[chunk], dstl)

            @pl.loop(0, nheads)
            def _(hh):
                pltpu.sync_copy(exT_hbm.at[chunk].at[hh], exl)

                @pl.loop(0, CP)
                def _(r):
                    for t in range(nseg):
                        acc.at[r].at[pl.ds(t * LANES, LANES)][...] = zseg

                def load_row(G, i):
                    segs = []
                    for t in range(nsg2):
                        g32 = G.at[i].at[pl.ds(t * 32, 32)][...]
                        a, bb = plsc.unpack(g32, format=plsc.PackFormat.INTERLEAVED)
                        segs.append(a)
                        segs.append(bb)
                    return segs

                def compute(b, G):
                    @pl.loop(0, KB // LANES)
                    def _(g):
                        base = b * KB + g * LANES
                        dl16 = dstl[pl.ds(base, LANES)]
                        ex16 = exl[pl.ds(base, LANES)]
                        segs = load_row(G, g * LANES)
                        for j in range(LANES):
                            d = dl16[j]
                            exs = ex16[j]
                            nxt = ([] if j + 1 == LANES
                                   else load_row(G, g * LANES + j + 1))
                            for t in range(nseg):
                                plsc.addupdate(
                                    acc.at[d].at[pl.ds(t * LANES, LANES)],
                                    exs * segs[t])
                            segs = nxt

                def start(b, G, sem):
                    pltpu.async_copy(
                        H_hbm.at[hh].at[srcl.at[pl.ds(b * KB, KB)]], G, sem)

                def wait(G, sem):
                    pltpu.make_async_copy(H_hbm.at[hh].at[pl.ds(0, KB)], G,
                                          sem).wait()

                @pl.when(nb > 0)
                def _():
                    start(0, G0, sem0)

                def pair(q, _):
                    b0 = 2 * q
                    b1 = b0 + 1

                    @pl.when(b1 < nb)
                    def _():
                        start(b1, G1, sem1)

                    wait(G0, sem0)
                    compute(b0, G0)

                    @pl.when(b1 + 1 < nb)
                    def _():
                        start(b1 + 1, G0, sem0)

                    @pl.when(b1 < nb)
                    def _():
                        wait(G1, sem1)
                        compute(b1, G1)
                    return 0

                lax.fori_loop(0, (nb + 1) // 2, pair, 0)

                pltpu.sync_copy(acc.at[pl.ds(0, C)],
                                acc_hbm.at[hh].at[pl.ds(chunk * C, C)])

    return k(H, src_c, dstloc_c, counts, exT)


BN = 512


def _tc_mm(Xin, W, a_src, a_dst):
    nheads, din, o = W.shape

    def body(x_ref, w_ref, asv_ref, adv_ref, H_ref, asrcT_ref, adstT_ref):
        h = pl.program_id(1)
        Hblk = jnp.dot(x_ref[...], w_ref[0], preferred_element_type=jnp.float32)
        Hperm = Hblk.reshape(BN, o // 32, 2, 16).swapaxes(2, 3).reshape(BN, o)
        H_ref[0] = Hperm.astype(jnp.bfloat16)
        asc = jnp.dot(Hblk, asv_ref[0, 0].reshape(o, 1),
                      preferred_element_type=jnp.float32)
        adc = jnp.dot(Hblk, adv_ref[0, 0].reshape(o, 1),
                      preferred_element_type=jnp.float32)
        lanes = lax.broadcasted_iota(jnp.int32, (BN, 16), 1)

        @pl.when(h == 0)
        def _():
            asrcT_ref[...] = jnp.zeros((BN, 16), jnp.float32)
            adstT_ref[...] = jnp.zeros((BN, 16), jnp.float32)

        asrcT_ref[...] = jnp.where(lanes == h, asc, asrcT_ref[...])
        adstT_ref[...] = jnp.where(lanes == h, adc, adstT_ref[...])

    return pl.pallas_call(
        body,
        grid=(NP // BN, nheads),
        in_specs=[
            pl.BlockSpec((BN, din), lambda nb, h: (nb, 0)),
            pl.BlockSpec((1, din, o), lambda nb, h: (h, 0, 0)),
            pl.BlockSpec((1, 1, o), lambda nb, h: (h, 0, 0)),
            pl.BlockSpec((1, 1, o), lambda nb, h: (h, 0, 0)),
        ],
        out_specs=[
            pl.BlockSpec((1, BN, o), lambda nb, h: (h, nb, 0)),
            pl.BlockSpec((BN, 16), lambda nb, h: (nb, 0)),
            pl.BlockSpec((BN, 16), lambda nb, h: (nb, 0)),
        ],
        out_shape=[
            jax.ShapeDtypeStruct((nheads, NP, o), jnp.bfloat16),
            jax.ShapeDtypeStruct((NP, 16), jnp.float32),
            jax.ShapeDtypeStruct((NP, 16), jnp.float32),
        ],
    )(Xin, W, a_src.reshape(nheads, 1, o), a_dst.reshape(nheads, 1, o))


def _tc_mid12(acc, den, skip=None):
    nheads, _, o = acc.shape

    def body(*refs):
        if skip is not None:
            acc_ref, den_ref, skip_ref, out_ref = refs
        else:
            acc_ref, den_ref, out_ref = refs
        cols = []
        for j in range(nheads):
            d = den_ref[...][:, j:j + 1] + 1e-16
            cols.append(acc_ref[j] / d)
        x = jnp.concatenate(cols, axis=1)
        if skip is not None:
            x = x + skip_ref[...]
        out_ref[...] = jnp.where(x > 0, x, jnp.exp(jnp.minimum(x, 0.0)) - 1.0)

    ins = [acc, den] + ([skip] if skip is not None else [])
    in_specs = [
        pl.BlockSpec((nheads, BN, o), lambda nb: (0, nb, 0)),
        pl.BlockSpec((BN, 16), lambda nb: (nb, 0)),
    ] + ([pl.BlockSpec((BN, nheads * o), lambda nb: (nb, 0))]
         if skip is not None else [])
    return pl.pallas_call(
        body,
        grid=(NP // BN,),
        in_specs=in_specs,
        out_specs=pl.BlockSpec((BN, nheads * o), lambda nb: (nb, 0)),
        out_shape=jax.ShapeDtypeStruct((NP, nheads * o), jnp.float32),
    )(*ins)


def _tc_mid3(acc, den, h2, Wskip):
    nheads, _, o = acc.shape

    def body(acc_ref, den_ref, h2_ref, wsk_ref, out_ref):
        tot = jnp.zeros((BN, o), jnp.float32)
        for j in range(nheads):
            d = den_ref[...][:, j:j + 1] + 1e-16
            tot = tot + acc_ref[j] / d
        tot = tot * (1.0 / nheads)
        tot = tot + jnp.dot(h2_ref[...], wsk_ref[...],
                            preferred_element_type=jnp.float32)
        out_ref[...] = tot

    return pl.pallas_call(
        body,
        grid=(NP // BN,),
        in_specs=[
            pl.BlockSpec((nheads, BN, o), lambda nb: (0, nb, 0)),
            pl.BlockSpec((BN, 16), lambda nb: (nb, 0)),
            pl.BlockSpec((BN, 1024), lambda nb: (nb, 0)),
            pl.BlockSpec((1024, o), lambda nb: (0, 0)),
        ],
        out_specs=pl.BlockSpec((BN, o), lambda nb: (nb, 0)),
        out_shape=jax.ShapeDtypeStruct((NP, o), jnp.float32),
    )(acc, den, h2, Wskip)


def kernel(X, edge_index, W1, a_src1, a_dst1, W2, a_src2, a_dst2,
           W3, a_src3, a_dst3, Wskip3):
    src = edge_index[0].astype(jnp.int32)
    dst = edge_index[1].astype(jnp.int32)
    srcp = jnp.concatenate([src, jnp.full((16,), N, jnp.int32)])
    dstp = jnp.concatenate([dst, jnp.full((16,), SENT_DST, jnp.int32)])
    Xp = jnp.pad(X, ((0, NP - N), (0, 0)))
    W3p = jnp.pad(W3, ((0, 0), (0, 0), (0, 7)))
    a_src3p = jnp.pad(a_src3, ((0, 0), (0, 7)))
    a_dst3p = jnp.pad(a_dst3, ((0, 0), (0, 7)))
    Wskip3p = jnp.pad(Wskip3, ((0, 0), (0, 7)))

    src_c, dstloc_c, counts = _sc_part(srcp, dstp)

    H1, as1, ad1 = _tc_mm(Xp, W1, a_src1, a_dst1)
    exT1, den1 = _sc_attn(src_c, dstloc_c, counts, as1, ad1)
    acc1 = _sc_aggr(H1, src_c, dstloc_c, counts, exT1, 4, 256)
    h1 = _tc_mid12(acc1, den1)

    H2, as2, ad2 = _tc_mm(h1, W2, a_src2, a_dst2)
    exT2, den2 = _sc_attn(src_c, dstloc_c, counts, as2, ad2)
    acc2 = _sc_aggr(H2, src_c, dstloc_c, counts, exT2, 4, 256)
    h2 = _tc_mid12(acc2, den2, skip=h1)

    H3, as3, ad3 = _tc_mm(h2, W3p, a_src3p, a_dst3p)
    exT3, den3 = _sc_attn(src_c, dstloc_c, counts, as3, ad3)
    acc3 = _sc_aggr(H3, src_c, dstloc_c, counts, exT3, 6, 128)
    out = _tc_mid3(acc3, den3, h2, Wskip3p)

    return out[:N, :121]


# speedup vs baseline: 1.6865x; 1.6865x over previous
import dataclasses
import functools

import jax
import jax.numpy as jnp
from jax import lax
from jax.experimental import pallas as pl
from jax.experimental.pallas import tpu as pltpu
from jax.experimental.pallas import tpu_sc as plsc

N = 10000
NP = 10240
E = 160000
NC, NS, LANES = 2, 16, 16
NW = NC * NS
NCH = 64
C = NP // NCH
CP = C + 8
CAP = 3072
EB = 4000
KA = 512
KB = 128
SENT_DST = 2 * N

_mesh = plsc.VectorSubcoreMesh(core_axis_name="c", subcore_axis_name="s")
_cp = pltpu.CompilerParams()
if "needs_layout_passes" in pltpu.CompilerParams.__dataclass_fields__:
    _cp = dataclasses.replace(_cp, needs_layout_passes=False)
if "use_tc_tiling_on_sc" in pltpu.CompilerParams.__dataclass_fields__:
    _cp = dataclasses.replace(_cp, use_tc_tiling_on_sc=False)

_IOTA = lambda: lax.iota(jnp.int32, LANES)


def _worker_id():
    return lax.axis_index("s") * NC + lax.axis_index("c")


def _sc_part(srcp, dstp):
    @functools.partial(
        pl.kernel,
        out_type=(
            jax.ShapeDtypeStruct((NCH, CAP), jnp.int32),
            jax.ShapeDtypeStruct((NCH, CAP), jnp.int32),
            jax.ShapeDtypeStruct((NCH * 16,), jnp.int32),
        ),
        mesh=_mesh,
        compiler_params=_cp,
        scratch_types=[
            pltpu.VMEM((EB,), jnp.int32),
            pltpu.VMEM((EB,), jnp.int32),
            pltpu.VMEM((CAP,), jnp.int32),
            pltpu.VMEM((CAP,), jnp.int32),
            pltpu.VMEM((CAP,), jnp.int32),
            pltpu.VMEM((CAP,), jnp.int32),
            pltpu.VMEM((16,), jnp.int32),
            pltpu.SemaphoreType.DMA,
            pltpu.SemaphoreType.DMA,
        ],
    )
    def k(srcp_hbm, dstp_hbm, src_c_hbm, dstloc_c_hbm, counts_hbm,
          dbuf0, dbuf1, eid0, eid1, srcv, dstv, cbuf, sem0, sem1):
        w = _worker_id()
        lo = w * 2 * C
        mid = lo + C
        hi = lo + 2 * C
        iota = _IOTA()
        nbat = E // EB

        def scan_batch(b, dbuf, carry):
            def group_body(g, ps):
                p0, p1 = ps
                d16 = dbuf[pl.ds(g * LANES, LANES)]
                eid16 = b * EB + g * LANES + iota
                mask0 = (d16 >= lo) & (d16 < mid)
                cum0 = plsc.cumsum(jnp.where(mask0, 1, 0))
                offs0 = jnp.minimum(p0 + cum0 - 1, CAP - 1)
                plsc.store_scatter(eid0, [offs0], eid16, mask=mask0)
                mask1 = (d16 >= mid) & (d16 < hi)
                cum1 = plsc.cumsum(jnp.where(mask1, 1, 0))
                offs1 = jnp.minimum(p1 + cum1 - 1, CAP - 1)
                plsc.store_scatter(eid1, [offs1], eid16, mask=mask1)
                return (jnp.minimum(p0 + cum0[15], CAP - 1),
                        jnp.minimum(p1 + cum1[15], CAP - 1))

            return lax.fori_loop(0, EB // LANES, group_body, carry)

        pltpu.async_copy(dstp_hbm.at[pl.ds(0, EB)], dbuf0, sem0)

        def batch_pair(q, carry):
            b0 = 2 * q
            b1 = b0 + 1

            pltpu.async_copy(dstp_hbm.at[pl.ds(b1 * EB, EB)], dbuf1, sem1)
            pltpu.make_async_copy(dstp_hbm.at[pl.ds(0, EB)], dbuf0, sem0).wait()
            carry = scan_batch(b0, dbuf0, carry)

            @pl.when(b1 + 1 < nbat)
            def _():
                pltpu.async_copy(dstp_hbm.at[pl.ds((b1 + 1) * EB, EB)],
                                 dbuf0, sem0)

            pltpu.make_async_copy(dstp_hbm.at[pl.ds(0, EB)], dbuf1, sem1).wait()
            carry = scan_batch(b1, dbuf1, carry)
            return carry

        cnt0, cnt1 = lax.fori_loop(0, nbat // 2, batch_pair, (0, 0))

        for cc, eidbuf, cnt, clo in ((0, eid0, cnt0, lo), (1, eid1, cnt1, mid)):
            chunk = w * 2 + cc

            @pl.loop(0, CAP // LANES)
            def _(g):
                pos = g * LANES + iota
                v = eidbuf[pl.ds(g * LANES, LANES)]
                eidbuf[pl.ds(g * LANES, LANES)] = jnp.where(pos >= cnt, E, v)

            pltpu.async_copy(srcp_hbm.at[eidbuf], srcv, sem0).wait()
            pltpu.async_copy(dstp_hbm.at[eidbuf], dstv, sem0).wait()

            @pl.loop(0, CAP // LANES)
            def _(g):
                v = dstv[pl.ds(g * LANES, LANES)]
                v = jnp.clip(jnp.minimum(v - clo, C), 0, C)
                dstv[pl.ds(g * LANES, LANES)] = v

            pltpu.sync_copy(srcv, src_c_hbm.at[chunk])
            pltpu.sync_copy(dstv, dstloc_c_hbm.at[chunk])
            cbuf[...] = jnp.where(iota == 0, cnt, 0)
            pltpu.sync_copy(cbuf, counts_hbm.at[pl.ds(chunk * 16, 16)])

    return k(srcp, dstp)


def _sc_attn(src_c, dstloc_c, counts, asrcT, adstT):
    @functools.partial(
        pl.kernel,
        out_type=(
            jax.ShapeDtypeStruct((NCH, 16, CAP), jnp.float32),
            jax.ShapeDtypeStruct((NP, 16), jnp.float32),
        ),
        mesh=_mesh,
        compiler_params=_cp,
        scratch_types=[
            pltpu.VMEM((16, CAP), jnp.float32),
            pltpu.VMEM((CP, 16), jnp.float32),
            pltpu.VMEM((CP, 16), jnp.float32),
            pltpu.VMEM((CP, 16), jnp.float32),
            pltpu.VMEM((KA, 16), jnp.float32),
            pltpu.VMEM((KA, 16), jnp.float32),
            pltpu.VMEM((CAP,), jnp.int32),
            pltpu.VMEM((CAP,), jnp.int32),
            pltpu.VMEM((16,), jnp.int32),
            pltpu.SemaphoreType.DMA,
            pltpu.SemaphoreType.DMA,
        ],
    )
    def k(src_c_hbm, dstloc_c_hbm, counts_hbm, asrcT_hbm, adstT_hbm,
          exT_hbm, den_hbm, eT, m, s, adst_loc, asg0, asg1, srcl, dstl,
          cbuf, sem0, sem1):
        w = _worker_id()
        iota = _IOTA()
        fz = jnp.zeros((LANES,), jnp.float32)

        for cc in range(2):
            chunk = w * 2 + cc
            pltpu.sync_copy(counts_hbm.at[pl.ds(chunk * 16, 16)], cbuf)
            cnt = cbuf[...][0]
            nb = (cnt + KA - 1) // KA

            pltpu.sync_copy(src_c_hbm.at[chunk], srcl)
            pltpu.sync_copy(dstloc_c_hbm.at[chunk], dstl)
            pltpu.sync_copy(adstT_hbm.at[pl.ds(chunk * C, C)],
                            adst_loc.at[pl.ds(0, C)])

            @pl.loop(0, CP)
            def _(r):
                m.at[r][...] = jnp.full((LANES,), -1e30, jnp.float32)
                s.at[r][...] = fz

            @pl.loop(C, CP)
            def _(r):
                adst_loc.at[r][...] = fz

            def p1_compute(b, asg):
                @pl.loop(0, KA // LANES)
                def _(g):
                    dl16 = dstl[pl.ds(b * KA + g * LANES, LANES)]
                    erows = []
                    for j in range(LANES):
                        i = g * LANES + j
                        e = asg.at[i][...] + adst_loc.at[dl16[j]][...]
                        erows.append(jnp.where(e > 0, e, 0.2 * e))
                    for j in range(LANES):
                        d = dl16[j]
                        m.at[d][...] = jnp.maximum(m.at[d][...], erows[j])
                    for j in range(LANES):
                        i = g * LANES + j
                        plsc.store_scatter(
                            eT, [iota, jnp.full((LANES,), b * KA + i,
                                                jnp.int32)], erows[j])

            def start(b, asg, sem):
                pltpu.async_copy(
                    asrcT_hbm.at[srcl.at[pl.ds(b * KA, KA)]], asg, sem)

            def wait(asg, sem):
                pltpu.make_async_copy(asrcT_hbm.at[pl.ds(0, KA)], asg,
                                      sem).wait()

            @pl.when(nb > 0)
            def _():
                start(0, asg0, sem0)

            def p1_pair(q, _):
                b0 = 2 * q
                b1 = b0 + 1

                @pl.when(b1 < nb)
                def _():
                    start(b1, asg1, sem1)

                wait(asg0, sem0)
                p1_compute(b0, asg0)

                @pl.when(b1 + 1 < nb)
                def _():
                    start(b1 + 1, asg0, sem0)

                @pl.when(b1 < nb)
                def _():
                    wait(asg1, sem1)
                    p1_compute(b1, asg1)
                return 0

            lax.fori_loop(0, (nb + 1) // 2, p1_pair, 0)

            def p2_batch(b, _):
                @pl.loop(0, KA // LANES)
                def _(g):
                    dl16 = dstl[pl.ds(b * KA + g * LANES, LANES)]
                    exs = []
                    for j in range(LANES):
                        i = g * LANES + j
                        colidx = jnp.full((LANES,), b * KA + i, jnp.int32)
                        erow = plsc.load_gather(eT, [iota, colidx])
                        exs.append(jnp.exp(erow - m.at[dl16[j]][...]))
                    for j in range(LANES):
                        plsc.addupdate(s.at[dl16[j]], exs[j])
                    for j in range(LANES):
                        i = g * LANES + j
                        plsc.store_scatter(
                            eT, [iota, jnp.full((LANES,), b * KA + i,
                                                jnp.int32)], exs[j])
                return 0

            lax.fori_loop(0, nb, p2_batch, 0)

            pltpu.sync_copy(s.at[pl.ds(0, C)],
                            den_hbm.at[pl.ds(chunk * C, C)])
            pltpu.sync_copy(eT, exT_hbm.at[chunk])

    return k(src_c, dstloc_c, counts, asrcT, adstT)


def _sc_aggr(H, src_c, dstloc_c, counts, exT, nheads, o):
    @functools.partial(
        pl.kernel,
        out_type=jax.ShapeDtypeStruct((nheads, NP, o), jnp.float32),
        mesh=_mesh,
        compiler_params=_cp,
        scratch_types=[
            pltpu.VMEM((CP, o), jnp.float32),
            pltpu.VMEM((KB, o), jnp.float32),
            pltpu.VMEM((KB, o), jnp.float32),
            pltpu.VMEM((CAP,), jnp.int32),
            pltpu.VMEM((CAP,), jnp.int32),
            pltpu.VMEM((CAP,), jnp.float32),
            pltpu.VMEM((16,), jnp.int32),
            pltpu.SemaphoreType.DMA,
            pltpu.SemaphoreType.DMA,
        ],
    )
    def k(H_hbm, src_c_hbm, dstloc_c_hbm, counts_hbm, exT_hbm, acc_hbm,
          acc, G0, G1, srcl, dstl, exl, cbuf, sem0, sem1):
        w = _worker_id()
        zseg = jnp.zeros((LANES,), jnp.float32)
        nseg = o // LANES

        @pl.loop(0, 2)
        def _(cc):
            chunk = w * 2 + cc
            pltpu.sync_copy(counts_hbm.at[pl.ds(chunk * 16, 16)], cbuf)
            cnt = cbuf[...][0]
            nb = (cnt + KB - 1) // KB
            pltpu.sync_copy(src_c_hbm.at[chunk], srcl)
            pltpu.sync_copy(dstloc_c_hbm.at[chunk], dstl)

            @pl.loop(0, nheads)
            def _(hh):
                pltpu.sync_copy(exT_hbm.at[chunk].at[hh], exl)

                @pl.loop(0, CP)
                def _(r):
                    for t in range(nseg):
                        acc.at[r].at[pl.ds(t * LANES, LANES)][...] = zseg

                def load_segs(G, i):
                    return [G.at[i].at[pl.ds(t * LANES, LANES)][...]
                            for t in range(nseg)]

                def compute(b, G):
                    @pl.loop(0, KB // LANES)
                    def _(g):
                        base = b * KB + g * LANES
                        dl16 = dstl[pl.ds(base, LANES)]
                        ex16 = exl[pl.ds(base, LANES)]
                        sA = load_segs(G, g * LANES)
                        sB = load_segs(G, g * LANES + 1)
                        for jp in range(0, LANES, 2):
                            d0 = dl16[jp]
                            e0 = ex16[jp]
                            d1 = dl16[jp + 1]
                            e1 = ex16[jp + 1]
                            nA = (load_segs(G, g * LANES + jp + 2)
                                  if jp + 2 < LANES else [])
                            nB = (load_segs(G, g * LANES + jp + 3)
                                  if jp + 3 < LANES else [])
                            for t in range(nseg):
                                plsc.addupdate(
                                    acc.at[d0].at[pl.ds(t * LANES, LANES)],
                                    e0 * sA[t])
                                plsc.addupdate(
                                    acc.at[d1].at[pl.ds(t * LANES, LANES)],
                                    e1 * sB[t])
                            sA, sB = nA, nB

                def start(b, G, sem):
                    pltpu.async_copy(
                        H_hbm.at[hh].at[srcl.at[pl.ds(b * KB, KB)]], G, sem)

                def wait(G, sem):
                    pltpu.make_async_copy(H_hbm.at[hh].at[pl.ds(0, KB)], G,
                                          sem).wait()

                @pl.when(nb > 0)
                def _():
                    start(0, G0, sem0)

                def pair(q, _):
                    b0 = 2 * q
                    b1 = b0 + 1

                    @pl.when(b1 < nb)
                    def _():
                        start(b1, G1, sem1)

                    wait(G0, sem0)
                    compute(b0, G0)

                    @pl.when(b1 + 1 < nb)
                    def _():
                        start(b1 + 1, G0, sem0)

                    @pl.when(b1 < nb)
                    def _():
                        wait(G1, sem1)
                        compute(b1, G1)
                    return 0

                lax.fori_loop(0, (nb + 1) // 2, pair, 0)

                pltpu.sync_copy(acc.at[pl.ds(0, C)],
                                acc_hbm.at[hh].at[pl.ds(chunk * C, C)])

    return k(H, src_c, dstloc_c, counts, exT)


BN = 512


def _tc_mm(Xin, W, a_src, a_dst):
    nheads, din, o = W.shape

    def body(x_ref, w_ref, asv_ref, adv_ref, H_ref, asrcT_ref, adstT_ref):
        h = pl.program_id(1)
        Hblk = jnp.dot(x_ref[...], w_ref[0], preferred_element_type=jnp.float32)
        H_ref[0] = Hblk
        asc = jnp.dot(Hblk, asv_ref[0, 0].reshape(o, 1),
                      preferred_element_type=jnp.float32)
        adc = jnp.dot(Hblk, adv_ref[0, 0].reshape(o, 1),
                      preferred_element_type=jnp.float32)
        lanes = lax.broadcasted_iota(jnp.int32, (BN, 16), 1)

        @pl.when(h == 0)
        def _():
            asrcT_ref[...] = jnp.zeros((BN, 16), jnp.float32)
            adstT_ref[...] = jnp.zeros((BN, 16), jnp.float32)

        asrcT_ref[...] = jnp.where(lanes == h, asc, asrcT_ref[...])
        adstT_ref[...] = jnp.where(lanes == h, adc, adstT_ref[...])

    return pl.pallas_call(
        body,
        grid=(NP // BN, nheads),
        in_specs=[
            pl.BlockSpec((BN, din), lambda nb, h: (nb, 0)),
            pl.BlockSpec((1, din, o), lambda nb, h: (h, 0, 0)),
            pl.BlockSpec((1, 1, o), lambda nb, h: (h, 0, 0)),
            pl.BlockSpec((1, 1, o), lambda nb, h: (h, 0, 0)),
        ],
        out_specs=[
            pl.BlockSpec((1, BN, o), lambda nb, h: (h, nb, 0)),
            pl.BlockSpec((BN, 16), lambda nb, h: (nb, 0)),
            pl.BlockSpec((BN, 16), lambda nb, h: (nb, 0)),
        ],
        out_shape=[
            jax.ShapeDtypeStruct((nheads, NP, o), jnp.float32),
            jax.ShapeDtypeStruct((NP, 16), jnp.float32),
            jax.ShapeDtypeStruct((NP, 16), jnp.float32),
        ],
    )(Xin, W, a_src.reshape(nheads, 1, o), a_dst.reshape(nheads, 1, o))


def _tc_mid12(acc, den, skip=None):
    nheads, _, o = acc.shape

    def body(*refs):
        if skip is not None:
            acc_ref, den_ref, skip_ref, out_ref = refs
        else:
            acc_ref, den_ref, out_ref = refs
        cols = []
        for j in range(nheads):
            d = den_ref[...][:, j:j + 1] + 1e-16
            cols.append(acc_ref[j] / d)
        x = jnp.concatenate(cols, axis=1)
        if skip is not None:
            x = x + skip_ref[...]
        out_ref[...] = jnp.where(x > 0, x, jnp.exp(jnp.minimum(x, 0.0)) - 1.0)

    ins = [acc, den] + ([skip] if skip is not None else [])
    in_specs = [
        pl.BlockSpec((nheads, BN, o), lambda nb: (0, nb, 0)),
        pl.BlockSpec((BN, 16), lambda nb: (nb, 0)),
    ] + ([pl.BlockSpec((BN, nheads * o), lambda nb: (nb, 0))]
         if skip is not None else [])
    return pl.pallas_call(
        body,
        grid=(NP // BN,),
        in_specs=in_specs,
        out_specs=pl.BlockSpec((BN, nheads * o), lambda nb: (nb, 0)),
        out_shape=jax.ShapeDtypeStruct((NP, nheads * o), jnp.float32),
    )(*ins)


def _tc_mid3(acc, den, h2, Wskip):
    nheads, _, o = acc.shape

    def body(acc_ref, den_ref, h2_ref, wsk_ref, out_ref):
        tot = jnp.zeros((BN, o), jnp.float32)
        for j in range(nheads):
            d = den_ref[...][:, j:j + 1] + 1e-16
            tot = tot + acc_ref[j] / d
        tot = tot * (1.0 / nheads)
        tot = tot + jnp.dot(h2_ref[...], wsk_ref[...],
                            preferred_element_type=jnp.float32)
        out_ref[...] = tot

    return pl.pallas_call(
        body,
        grid=(NP // BN,),
        in_specs=[
            pl.BlockSpec((nheads, BN, o), lambda nb: (0, nb, 0)),
            pl.BlockSpec((BN, 16), lambda nb: (nb, 0)),
            pl.BlockSpec((BN, 1024), lambda nb: (nb, 0)),
            pl.BlockSpec((1024, o), lambda nb: (0, 0)),
        ],
        out_specs=pl.BlockSpec((BN, o), lambda nb: (nb, 0)),
        out_shape=jax.ShapeDtypeStruct((NP, o), jnp.float32),
    )(acc, den, h2, Wskip)


def kernel(X, edge_index, W1, a_src1, a_dst1, W2, a_src2, a_dst2,
           W3, a_src3, a_dst3, Wskip3):
    src = edge_index[0].astype(jnp.int32)
    dst = edge_index[1].astype(jnp.int32)
    srcp = jnp.concatenate([src, jnp.full((16,), N, jnp.int32)])
    dstp = jnp.concatenate([dst, jnp.full((16,), SENT_DST, jnp.int32)])
    Xp = jnp.pad(X, ((0, NP - N), (0, 0)))
    W3p = jnp.pad(W3, ((0, 0), (0, 0), (0, 7)))
    a_src3p = jnp.pad(a_src3, ((0, 0), (0, 7)))
    a_dst3p = jnp.pad(a_dst3, ((0, 0), (0, 7)))
    Wskip3p = jnp.pad(Wskip3, ((0, 0), (0, 7)))

    src_c, dstloc_c, counts = _sc_part(srcp, dstp)

    H1, as1, ad1 = _tc_mm(Xp, W1, a_src1, a_dst1)
    exT1, den1 = _sc_attn(src_c, dstloc_c, counts, as1, ad1)
    acc1 = _sc_aggr(H1, src_c, dstloc_c, counts, exT1, 4, 256)
    h1 = _tc_mid12(acc1, den1)

    H2, as2, ad2 = _tc_mm(h1, W2, a_src2, a_dst2)
    exT2, den2 = _sc_attn(src_c, dstloc_c, counts, as2, ad2)
    acc2 = _sc_aggr(H2, src_c, dstloc_c, counts, exT2, 4, 256)
    h2 = _tc_mid12(acc2, den2, skip=h1)

    H3, as3, ad3 = _tc_mm(h2, W3p, a_src3p, a_dst3p)
    exT3, den3 = _sc_attn(src_c, dstloc_c, counts, as3, ad3)
    acc3 = _sc_aggr(H3, src_c, dstloc_c, counts, exT3, 6, 128)
    out = _tc_mid3(acc3, den3, h2, Wskip3p)

    return out[:N, :121]


# speedup vs baseline: 1.9710x; 1.1687x over previous
import dataclasses
import functools

import jax
import jax.numpy as jnp
from jax import lax
from jax.experimental import pallas as pl
from jax.experimental.pallas import tpu as pltpu
from jax.experimental.pallas import tpu_sc as plsc

N = 10000
NP = 10240
E = 160000
NC, NS, LANES = 2, 16, 16
NW = NC * NS
NCH = 64
C = NP // NCH
CP = C + 8
CAP = 3072
EB = 4000
KA = 512
KB = 128
SENT_DST = 2 * N

_mesh = plsc.VectorSubcoreMesh(core_axis_name="c", subcore_axis_name="s")
_cp = pltpu.CompilerParams()
if "needs_layout_passes" in pltpu.CompilerParams.__dataclass_fields__:
    _cp = dataclasses.replace(_cp, needs_layout_passes=False)
if "use_tc_tiling_on_sc" in pltpu.CompilerParams.__dataclass_fields__:
    _cp = dataclasses.replace(_cp, use_tc_tiling_on_sc=False)

_IOTA = lambda: lax.iota(jnp.int32, LANES)


def _worker_id():
    return lax.axis_index("s") * NC + lax.axis_index("c")


def _sc_part(srcp, dstp):
    @functools.partial(
        pl.kernel,
        out_type=(
            jax.ShapeDtypeStruct((NCH, CAP), jnp.int32),
            jax.ShapeDtypeStruct((NCH, CAP), jnp.int32),
            jax.ShapeDtypeStruct((NCH * 16,), jnp.int32),
        ),
        mesh=_mesh,
        compiler_params=_cp,
        scratch_types=[
            pltpu.VMEM((EB,), jnp.int32),
            pltpu.VMEM((EB,), jnp.int32),
            pltpu.VMEM((CAP,), jnp.int32),
            pltpu.VMEM((CAP,), jnp.int32),
            pltpu.VMEM((CAP,), jnp.int32),
            pltpu.VMEM((CAP,), jnp.int32),
            pltpu.VMEM((16,), jnp.int32),
            pltpu.SemaphoreType.DMA,
            pltpu.SemaphoreType.DMA,
        ],
    )
    def k(srcp_hbm, dstp_hbm, src_c_hbm, dstloc_c_hbm, counts_hbm,
          dbuf0, dbuf1, eid0, eid1, srcv, dstv, cbuf, sem0, sem1):
        w = _worker_id()
        lo = w * 2 * C
        mid = lo + C
        hi = lo + 2 * C
        iota = _IOTA()
        nbat = E // EB

        def scan_batch(b, dbuf, carry):
            def group_body(g, ps):
                p0, p1 = ps
                d16 = dbuf[pl.ds(g * LANES, LANES)]
                eid16 = b * EB + g * LANES + iota
                mask0 = (d16 >= lo) & (d16 < mid)
                cum0 = plsc.cumsum(jnp.where(mask0, 1, 0))
                offs0 = jnp.minimum(p0 + cum0 - 1, CAP - 1)
                plsc.store_scatter(eid0, [offs0], eid16, mask=mask0)
                mask1 = (d16 >= mid) & (d16 < hi)
                cum1 = plsc.cumsum(jnp.where(mask1, 1, 0))
                offs1 = jnp.minimum(p1 + cum1 - 1, CAP - 1)
                plsc.store_scatter(eid1, [offs1], eid16, mask=mask1)
                return (jnp.minimum(p0 + cum0[15], CAP - 1),
                        jnp.minimum(p1 + cum1[15], CAP - 1))

            return lax.fori_loop(0, EB // LANES, group_body, carry)

        pltpu.async_copy(dstp_hbm.at[pl.ds(0, EB)], dbuf0, sem0)

        def batch_pair(q, carry):
            b0 = 2 * q
            b1 = b0 + 1

            pltpu.async_copy(dstp_hbm.at[pl.ds(b1 * EB, EB)], dbuf1, sem1)
            pltpu.make_async_copy(dstp_hbm.at[pl.ds(0, EB)], dbuf0, sem0).wait()
            carry = scan_batch(b0, dbuf0, carry)

            @pl.when(b1 + 1 < nbat)
            def _():
                pltpu.async_copy(dstp_hbm.at[pl.ds((b1 + 1) * EB, EB)],
                                 dbuf0, sem0)

            pltpu.make_async_copy(dstp_hbm.at[pl.ds(0, EB)], dbuf1, sem1).wait()
            carry = scan_batch(b1, dbuf1, carry)
            return carry

        cnt0, cnt1 = lax.fori_loop(0, nbat // 2, batch_pair, (0, 0))

        for cc, eidbuf, cnt, clo in ((0, eid0, cnt0, lo), (1, eid1, cnt1, mid)):
            chunk = w * 2 + cc

            @pl.loop(0, CAP // LANES)
            def _(g):
                pos = g * LANES + iota
                v = eidbuf[pl.ds(g * LANES, LANES)]
                eidbuf[pl.ds(g * LANES, LANES)] = jnp.where(pos >= cnt, E, v)

            pltpu.async_copy(srcp_hbm.at[eidbuf], srcv, sem0).wait()
            pltpu.async_copy(dstp_hbm.at[eidbuf], dstv, sem0).wait()

            @pl.loop(0, CAP // LANES)
            def _(g):
                v = dstv[pl.ds(g * LANES, LANES)]
                v = jnp.clip(jnp.minimum(v - clo, C), 0, C)
                dstv[pl.ds(g * LANES, LANES)] = v

            pltpu.sync_copy(srcv, src_c_hbm.at[chunk])
            pltpu.sync_copy(dstv, dstloc_c_hbm.at[chunk])
            cbuf[...] = jnp.where(iota == 0, cnt, 0)
            pltpu.sync_copy(cbuf, counts_hbm.at[pl.ds(chunk * 16, 16)])

    return k(srcp, dstp)


def _sc_attn(src_c, dstloc_c, counts, asrcT, adstT):
    @functools.partial(
        pl.kernel,
        out_type=(
            jax.ShapeDtypeStruct((NCH, 16, CAP), jnp.float32),
            jax.ShapeDtypeStruct((NP, 16), jnp.float32),
        ),
        mesh=_mesh,
        compiler_params=_cp,
        scratch_types=[
            pltpu.VMEM((16, CAP), jnp.float32),
            pltpu.VMEM((CP, 16), jnp.float32),
            pltpu.VMEM((CP, 16), jnp.float32),
            pltpu.VMEM((CP, 16), jnp.float32),
            pltpu.VMEM((KA, 16), jnp.float32),
            pltpu.VMEM((KA, 16), jnp.float32),
            pltpu.VMEM((CAP,), jnp.int32),
            pltpu.VMEM((CAP,), jnp.int32),
            pltpu.VMEM((16,), jnp.int32),
            pltpu.SemaphoreType.DMA,
            pltpu.SemaphoreType.DMA,
        ],
    )
    def k(src_c_hbm, dstloc_c_hbm, counts_hbm, asrcT_hbm, adstT_hbm,
          exT_hbm, den_hbm, eT, m, s, adst_loc, asg0, asg1, srcl, dstl,
          cbuf, sem0, sem1):
        w = _worker_id()
        iota = _IOTA()
        fz = jnp.zeros((LANES,), jnp.float32)

        for cc in range(2):
            chunk = w * 2 + cc
            pltpu.sync_copy(counts_hbm.at[pl.ds(chunk * 16, 16)], cbuf)
            cnt = cbuf[...][0]
            nb = (cnt + KA - 1) // KA

            pltpu.sync_copy(src_c_hbm.at[chunk], srcl)
            pltpu.sync_copy(dstloc_c_hbm.at[chunk], dstl)
            pltpu.sync_copy(adstT_hbm.at[pl.ds(chunk * C, C)],
                            adst_loc.at[pl.ds(0, C)])

            @pl.loop(0, CP)
            def _(r):
                m.at[r][...] = jnp.full((LANES,), -1e30, jnp.float32)
                s.at[r][...] = fz

            @pl.loop(C, CP)
            def _(r):
                adst_loc.at[r][...] = fz

            def p1_compute(b, asg):
                @pl.loop(0, KA // LANES)
                def _(g):
                    dl16 = dstl[pl.ds(b * KA + g * LANES, LANES)]
                    erows = []
                    for j in range(LANES):
                        i = g * LANES + j
                        e = asg.at[i][...] + adst_loc.at[dl16[j]][...]
                        erows.append(jnp.where(e > 0, e, 0.2 * e))
                    for j in range(LANES):
                        d = dl16[j]
                        m.at[d][...] = jnp.maximum(m.at[d][...], erows[j])
                    for j in range(LANES):
                        i = g * LANES + j
                        plsc.store_scatter(
                            eT, [iota, jnp.full((LANES,), b * KA + i,
                                                jnp.int32)], erows[j])

            def start(b, asg, sem):
                pltpu.async_copy(
                    asrcT_hbm.at[srcl.at[pl.ds(b * KA, KA)]], asg, sem)

            def wait(asg, sem):
                pltpu.make_async_copy(asrcT_hbm.at[pl.ds(0, KA)], asg,
                                      sem).wait()

            @pl.when(nb > 0)
            def _():
                start(0, asg0, sem0)

            def p1_pair(q, _):
                b0 = 2 * q
                b1 = b0 + 1

                @pl.when(b1 < nb)
                def _():
                    start(b1, asg1, sem1)

                wait(asg0, sem0)
                p1_compute(b0, asg0)

                @pl.when(b1 + 1 < nb)
                def _():
                    start(b1 + 1, asg0, sem0)

                @pl.when(b1 < nb)
                def _():
                    wait(asg1, sem1)
                    p1_compute(b1, asg1)
                return 0

            lax.fori_loop(0, (nb + 1) // 2, p1_pair, 0)

            def p2_batch(b, _):
                @pl.loop(0, KA // LANES)
                def _(g):
                    dl16 = dstl[pl.ds(b * KA + g * LANES, LANES)]
                    exs = []
                    for j in range(LANES):
                        i = g * LANES + j
                        colidx = jnp.full((LANES,), b * KA + i, jnp.int32)
                        erow = plsc.load_gather(eT, [iota, colidx])
                        exs.append(jnp.exp(erow - m.at[dl16[j]][...]))
                    for j in range(LANES):
                        plsc.addupdate(s.at[dl16[j]], exs[j])
                    for j in range(LANES):
                        i = g * LANES + j
                        plsc.store_scatter(
                            eT, [iota, jnp.full((LANES,), b * KA + i,
                                                jnp.int32)], exs[j])
                return 0

            lax.fori_loop(0, nb, p2_batch, 0)

            pltpu.sync_copy(s.at[pl.ds(0, C)],
                            den_hbm.at[pl.ds(chunk * C, C)])
            pltpu.sync_copy(eT, exT_hbm.at[chunk])

    return k(src_c, dstloc_c, counts, asrcT, adstT)


def _sc_aggr(H, src_c, dstloc_c, counts, exT, nheads, o):
    @functools.partial(
        pl.kernel,
        out_type=jax.ShapeDtypeStruct((nheads, NP, o), jnp.float32),
        mesh=_mesh,
        compiler_params=_cp,
        scratch_types=[
            pltpu.VMEM((CP, o), jnp.float32),
            pltpu.VMEM((KB, o), jnp.bfloat16),
            pltpu.VMEM((KB, o), jnp.bfloat16),
            pltpu.VMEM((CAP,), jnp.int32),
            pltpu.VMEM((CAP,), jnp.int32),
            pltpu.VMEM((CAP,), jnp.float32),
            pltpu.VMEM((16,), jnp.int32),
            pltpu.SemaphoreType.DMA,
            pltpu.SemaphoreType.DMA,
        ],
    )
    def k(H_hbm, src_c_hbm, dstloc_c_hbm, counts_hbm, exT_hbm, acc_hbm,
          acc, G0, G1, srcl, dstl, exl, cbuf, sem0, sem1):
        w = _worker_id()
        zseg = jnp.zeros((LANES,), jnp.float32)
        nseg = o // LANES

        @pl.loop(0, 2)
        def _(cc):
            chunk = w * 2 + cc
            pltpu.sync_copy(counts_hbm.at[pl.ds(chunk * 16, 16)], cbuf)
            cnt = cbuf[...][0]
            nb = (cnt + KB - 1) // KB
            pltpu.sync_copy(src_c_hbm.at[chunk], srcl)
            pltpu.sync_copy(dstloc_c_hbm.at[chunk], dstl)

            @pl.loop(0, nheads)
            def _(hh):
                pltpu.sync_copy(exT_hbm.at[chunk].at[hh], exl)

                @pl.loop(0, CP)
                def _(r):
                    for t in range(nseg):
                        acc.at[r].at[pl.ds(t * LANES, LANES)][...] = zseg

                def load_segs(G, i):
                    segs = []
                    for t in range(o // 32):
                        g32 = G.at[i].at[pl.ds(t * 32, 32)][...]
                        a, bb = plsc.unpack(
                            g32, format=plsc.PackFormat.INTERLEAVED)
                        segs.append(a)
                        segs.append(bb)
                    return segs

                def compute(b, G):
                    @pl.loop(0, KB // LANES)
                    def _(g):
                        base = b * KB + g * LANES
                        dl16 = dstl[pl.ds(base, LANES)]
                        ex16 = exl[pl.ds(base, LANES)]
                        sA = load_segs(G, g * LANES)
                        sB = load_segs(G, g * LANES + 1)
                        for jp in range(0, LANES, 2):
                            d0 = dl16[jp]
                            e0 = ex16[jp]
                            d1 = dl16[jp + 1]
                            e1 = ex16[jp + 1]
                            nA = (load_segs(G, g * LANES + jp + 2)
                                  if jp + 2 < LANES else [])
                            nB = (load_segs(G, g * LANES + jp + 3)
                                  if jp + 3 < LANES else [])
                            for t in range(nseg):
                                plsc.addupdate(
                                    acc.at[d0].at[pl.ds(t * LANES, LANES)],
                                    e0 * sA[t])
                                plsc.addupdate(
                                    acc.at[d1].at[pl.ds(t * LANES, LANES)],
                                    e1 * sB[t])
                            sA, sB = nA, nB

                def start(b, G, sem):
                    pltpu.async_copy(
                        H_hbm.at[hh].at[srcl.at[pl.ds(b * KB, KB)]], G, sem)

                def wait(G, sem):
                    pltpu.make_async_copy(H_hbm.at[hh].at[pl.ds(0, KB)], G,
                                          sem).wait()

                @pl.when(nb > 0)
                def _():
                    start(0, G0, sem0)

                def pair(q, _):
                    b0 = 2 * q
                    b1 = b0 + 1

                    @pl.when(b1 < nb)
                    def _():
                        start(b1, G1, sem1)

                    wait(G0, sem0)
                    compute(b0, G0)

                    @pl.when(b1 + 1 < nb)
                    def _():
                        start(b1 + 1, G0, sem0)

                    @pl.when(b1 < nb)
                    def _():
                        wait(G1, sem1)
                        compute(b1, G1)
                    return 0

                lax.fori_loop(0, (nb + 1) // 2, pair, 0)

                pltpu.sync_copy(acc.at[pl.ds(0, C)],
                                acc_hbm.at[hh].at[pl.ds(chunk * C, C)])

    return k(H, src_c, dstloc_c, counts, exT)


BN = 512


def _tc_mm(Xin, W, a_src, a_dst):
    nheads, din, o = W.shape

    def body(x_ref, w_ref, asv_ref, adv_ref, H_ref, asrcT_ref, adstT_ref):
        h = pl.program_id(1)
        Hblk = jnp.dot(x_ref[...], w_ref[0], preferred_element_type=jnp.float32)
        H_ref[0] = Hblk.astype(jnp.bfloat16)
        asc = jnp.dot(Hblk, asv_ref[0, 0].reshape(o, 1),
                      preferred_element_type=jnp.float32)
        adc = jnp.dot(Hblk, adv_ref[0, 0].reshape(o, 1),
                      preferred_element_type=jnp.float32)
        lanes = lax.broadcasted_iota(jnp.int32, (BN, 16), 1)

        @pl.when(h == 0)
        def _():
            asrcT_ref[...] = jnp.zeros((BN, 16), jnp.float32)
            adstT_ref[...] = jnp.zeros((BN, 16), jnp.float32)

        asrcT_ref[...] = jnp.where(lanes == h, asc, asrcT_ref[...])
        adstT_ref[...] = jnp.where(lanes == h, adc, adstT_ref[...])

    return pl.pallas_call(
        body,
        grid=(NP // BN, nheads),
        in_specs=[
            pl.BlockSpec((BN, din), lambda nb, h: (nb, 0)),
            pl.BlockSpec((1, din, o), lambda nb, h: (h, 0, 0)),
            pl.BlockSpec((1, 1, o), lambda nb, h: (h, 0, 0)),
            pl.BlockSpec((1, 1, o), lambda nb, h: (h, 0, 0)),
        ],
        out_specs=[
            pl.BlockSpec((1, BN, o), lambda nb, h: (h, nb, 0)),
            pl.BlockSpec((BN, 16), lambda nb, h: (nb, 0)),
            pl.BlockSpec((BN, 16), lambda nb, h: (nb, 0)),
        ],
        out_shape=[
            jax.ShapeDtypeStruct((nheads, NP, o), jnp.bfloat16),
            jax.ShapeDtypeStruct((NP, 16), jnp.float32),
            jax.ShapeDtypeStruct((NP, 16), jnp.float32),
        ],
    )(Xin, W, a_src.reshape(nheads, 1, o), a_dst.reshape(nheads, 1, o))


def _tc_mid12(acc, den, skip=None):
    nheads, _, o = acc.shape

    def body(*refs):
        if skip is not None:
            acc_ref, den_ref, skip_ref, out_ref = refs
        else:
            acc_ref, den_ref, out_ref = refs
        cols = []
        for j in range(nheads):
            d = den_ref[...][:, j:j + 1] + 1e-16
            cols.append(acc_ref[j] / d)
        x = jnp.concatenate(cols, axis=1)
        if skip is not None:
            x = x + skip_ref[...]
        out_ref[...] = jnp.where(x > 0, x, jnp.exp(jnp.minimum(x, 0.0)) - 1.0)

    ins = [acc, den] + ([skip] if skip is not None else [])
    in_specs = [
        pl.BlockSpec((nheads, BN, o), lambda nb: (0, nb, 0)),
        pl.BlockSpec((BN, 16), lambda nb: (nb, 0)),
    ] + ([pl.BlockSpec((BN, nheads * o), lambda nb: (nb, 0))]
         if skip is not None else [])
    return pl.pallas_call(
        body,
        grid=(NP // BN,),
        in_specs=in_specs,
        out_specs=pl.BlockSpec((BN, nheads * o), lambda nb: (nb, 0)),
        out_shape=jax.ShapeDtypeStruct((NP, nheads * o), jnp.float32),
    )(*ins)


def _tc_mid3(acc, den, h2, Wskip):
    nheads, _, o = acc.shape

    def body(acc_ref, den_ref, h2_ref, wsk_ref, out_ref):
        tot = jnp.zeros((BN, o), jnp.float32)
        for j in range(nheads):
            d = den_ref[...][:, j:j + 1] + 1e-16
            tot = tot + acc_ref[j] / d
        tot = tot * (1.0 / nheads)
        tot = tot + jnp.dot(h2_ref[...], wsk_ref[...],
                            preferred_element_type=jnp.float32)
        out_ref[...] = tot

    return pl.pallas_call(
        body,
        grid=(NP // BN,),
        in_specs=[
            pl.BlockSpec((nheads, BN, o), lambda nb: (0, nb, 0)),
            pl.BlockSpec((BN, 16), lambda nb: (nb, 0)),
            pl.BlockSpec((BN, 1024), lambda nb: (nb, 0)),
            pl.BlockSpec((1024, o), lambda nb: (0, 0)),
        ],
        out_specs=pl.BlockSpec((BN, o), lambda nb: (nb, 0)),
        out_shape=jax.ShapeDtypeStruct((NP, o), jnp.float32),
    )(acc, den, h2, Wskip)


def _icl(x):
    o = x.shape[-1]
    lead = x.shape[:-1]
    return (x.reshape(*lead, o // 32, 2, 16)
            .swapaxes(-1, -2)
            .reshape(*lead, o))


def kernel(X, edge_index, W1, a_src1, a_dst1, W2, a_src2, a_dst2,
           W3, a_src3, a_dst3, Wskip3):
    src = edge_index[0].astype(jnp.int32)
    dst = edge_index[1].astype(jnp.int32)
    srcp = jnp.concatenate([src, jnp.full((16,), N, jnp.int32)])
    dstp = jnp.concatenate([dst, jnp.full((16,), SENT_DST, jnp.int32)])
    Xp = jnp.pad(X, ((0, NP - N), (0, 0)))
    W1 = _icl(W1)
    a_src1 = _icl(a_src1)
    a_dst1 = _icl(a_dst1)
    W2 = _icl(W2)
    a_src2 = _icl(a_src2)
    a_dst2 = _icl(a_dst2)
    W3p = _icl(jnp.pad(W3, ((0, 0), (0, 0), (0, 7))))
    a_src3p = _icl(jnp.pad(a_src3, ((0, 0), (0, 7))))
    a_dst3p = _icl(jnp.pad(a_dst3, ((0, 0), (0, 7))))
    Wskip3p = jnp.pad(Wskip3, ((0, 0), (0, 7)))

    src_c, dstloc_c, counts = _sc_part(srcp, dstp)

    H1, as1, ad1 = _tc_mm(Xp, W1, a_src1, a_dst1)
    exT1, den1 = _sc_attn(src_c, dstloc_c, counts, as1, ad1)
    acc1 = _sc_aggr(H1, src_c, dstloc_c, counts, exT1, 4, 256)
    h1 = _tc_mid12(acc1, den1)

    H2, as2, ad2 = _tc_mm(h1, W2, a_src2, a_dst2)
    exT2, den2 = _sc_attn(src_c, dstloc_c, counts, as2, ad2)
    acc2 = _sc_aggr(H2, src_c, dstloc_c, counts, exT2, 4, 256)
    h2 = _tc_mid12(acc2, den2, skip=h1)

    H3, as3, ad3 = _tc_mm(h2, W3p, a_src3p, a_dst3p)
    exT3, den3 = _sc_attn(src_c, dstloc_c, counts, as3, ad3)
    acc3 = _sc_aggr(H3, src_c, dstloc_c, counts, exT3, 6, 128)
    out = _tc_mid3(acc3, den3, h2, Wskip3p)

    return out[:N, :121]
